# baseline (speedup 1.0000x reference)
import functools
import jax, jax.numpy as jnp
from jax import lax
from jax.experimental import pallas as pl
from jax.experimental.pallas import tpu as pltpu
from jax.experimental.pallas import tpu_sc as plsc

@functools.partial(
    pl.kernel,
    mesh=plsc.VectorSubcoreMesh(core_axis_name="c", subcore_axis_name="s"),
    compiler_params=pltpu.CompilerParams(use_tc_tiling_on_sc=True),
    out_type=jax.ShapeDtypeStruct((4096, 200, 64), jnp.float32),
    scratch_types=[
        pltpu.VMEM((200, 128), jnp.int32),
        pltpu.SemaphoreType.DMA,
        pltpu.SemaphoreType.DMA,
    ],
)
def _k(xt_hbm, table_hbm, out_hbm, idx_v, g0, g1):
    gsem = (g0, g1)
    wid = lax.axis_index("s") * 2 + lax.axis_index("c")
    b0 = wid * 128
    pltpu.sync_copy(xt_hbm.at[:, pl.ds(b0, 128)], idx_v)

    def start_gathers(t, slot):
        for bg in range(8):
            vec = idx_v[t, pl.ds(bg * 16, 16)]
            for k in range(16):
                vk = vec[k]
                pltpu.async_copy(
                    table_hbm.at[vk], out_hbm.at[b0 + bg * 16 + k, t], gsem[slot]
                )

    def wait_gathers(t, slot):
        pltpu.make_async_copy(
            table_hbm.at[pl.ds(0, 128), :], out_hbm.at[pl.ds(b0, 128), t], gsem[slot]
        ).wait()

    start_gathers(0, 0)
    start_gathers(1, 1)

    def pair_body(p, carry):
        for slot in (0, 1):
            t = p * 2 + slot
            wait_gathers(t, slot)

            @pl.when(t + 2 < 200)
            def _(slot=slot, t=t):
                start_gathers(t + 2, slot)
        return carry

    lax.fori_loop(0, 100, pair_body, 0)

def kernel(x, table):
    xt = x.T.astype(jnp.int32)
    return _k(xt, table)


# speedup vs baseline: 11.4142x; 11.4142x over previous
import functools
import jax, jax.numpy as jnp
from jax import lax
from jax.experimental import pallas as pl
from jax.experimental.pallas import tpu as pltpu
from jax.experimental.pallas import tpu_sc as plsc

HB = 2048
BBLK = HB // 32

@functools.partial(
    pl.kernel,
    mesh=plsc.VectorSubcoreMesh(core_axis_name="c", subcore_axis_name="s"),
    compiler_params=pltpu.CompilerParams(use_tc_tiling_on_sc=True),
    out_type=jax.ShapeDtypeStruct((HB, 200, 64), jnp.float32),
    scratch_types=[
        pltpu.VMEM((200, 128), jnp.int32),
        pltpu.VMEM((BBLK, 64), jnp.float32),
        pltpu.VMEM((BBLK, 64), jnp.float32),
        pltpu.SemaphoreType.DMA,
        pltpu.SemaphoreType.DMA,
        pltpu.SemaphoreType.DMA,
        pltpu.SemaphoreType.DMA,
    ],
)
def _k(xt_hbm, table_hbm, out_hbm, idx_v, rows0, rows1, g0, g1, s0, s1):
    rows = (rows0, rows1)
    gsem = (g0, g1)
    ssem = (s0, s1)
    wid = lax.axis_index("s") * 2 + lax.axis_index("c")
    b0 = wid * BBLK
    xoff = (wid % 2) * BBLK
    pltpu.sync_copy(xt_hbm.at[:, pl.ds((wid // 2) * 128, 128)], idx_v)

    def start_gathers(t, slot):
        for bg in range(BBLK // 16):
            vec = idx_v[t, pl.ds(xoff + bg * 16, 16)]
            for k in range(16):
                vk = vec[k]
                pltpu.async_copy(
                    table_hbm.at[vk], rows[slot].at[bg * 16 + k], gsem[slot]
                )

    def wait_gathers(t, slot):
        pltpu.make_async_copy(
            table_hbm.at[pl.ds(0, BBLK), :], rows[slot], gsem[slot]
        ).wait()

    def start_store(t, slot):
        pltpu.async_copy(rows[slot], out_hbm.at[pl.ds(b0, BBLK), t], ssem[slot])

    def wait_store(t, slot):
        pltpu.make_async_copy(
            rows[slot], out_hbm.at[pl.ds(b0, BBLK), t], ssem[slot]
        ).wait()

    start_gathers(0, 0)
    start_gathers(1, 1)

    def pair_body(p, carry):
        for slot in (0, 1):
            t = p * 2 + slot
            wait_gathers(t, slot)
            start_store(t, slot)

            @pl.when(t + 2 < 200)
            def _(slot=slot, t=t):
                wait_store(t, slot)
                start_gathers(t + 2, slot)
        return carry

    lax.fori_loop(0, 100, pair_body, 0)
    wait_store(198, 0)
    wait_store(199, 1)

def kernel(x, table):
    xt = x.T.astype(jnp.int32)
    out_a = _k(xt[:, :HB], table)
    out_b = _k(xt[:, HB:], table)
    return jnp.concatenate([out_a, out_b], axis=0)


# speedup vs baseline: 14.3283x; 1.2553x over previous
import functools

import jax
import jax.numpy as jnp
from jax import lax
from jax.experimental import pallas as pl
from jax.experimental.pallas import tpu as pltpu
from jax.experimental.pallas import tpu_sc as plsc

B_ROWS = 4096
SEQ = 200
EMBED = 64

NC = 2
NS = 16
NW = NC * NS
BBLK = B_ROWS // NW
L = 16
NSLOT = 4


@functools.partial(
    pl.kernel,
    mesh=plsc.VectorSubcoreMesh(core_axis_name="c", subcore_axis_name="s"),
    compiler_params=pltpu.CompilerParams(use_tc_tiling_on_sc=True),
    out_type=jax.ShapeDtypeStruct((B_ROWS, SEQ, EMBED), jnp.float32),
    scratch_types=(
        [pltpu.VMEM((SEQ, BBLK), jnp.int32)]
        + [pltpu.VMEM((BBLK, EMBED), jnp.float32) for _ in range(NSLOT)]
        + [pltpu.SemaphoreType.DMA for _ in range(2 * NSLOT)]
    ),
)
def _emb_lookup(xt_hbm, table_hbm, out_hbm, idx_v, *rest):
    rows = rest[0:NSLOT]
    gsem = rest[NSLOT : 2 * NSLOT]
    ssem = rest[2 * NSLOT :]
    wid = lax.axis_index("s") * NC + lax.axis_index("c")
    b0 = wid * BBLK

    pltpu.sync_copy(xt_hbm.at[:, pl.ds(b0, BBLK)], idx_v)

    def start_gathers(t, slot):
        for bg in range(BBLK // L):
            vec = idx_v[t, pl.ds(bg * L, L)]
            for k in range(L):
                vk = vec[k]
                pltpu.async_copy(
                    table_hbm.at[vk], rows[slot].at[bg * L + k], gsem[slot]
                )

    def wait_gathers(t, slot):
        pltpu.make_async_copy(
            table_hbm.at[pl.ds(0, BBLK), :], rows[slot], gsem[slot]
        ).wait()

    def start_store(t, slot):
        pltpu.async_copy(rows[slot], out_hbm.at[pl.ds(b0, BBLK), t], ssem[slot])

    def wait_store(t, slot):
        pltpu.make_async_copy(
            rows[slot], out_hbm.at[pl.ds(b0, BBLK), t], ssem[slot]
        ).wait()

    for t in range(NSLOT):
        start_gathers(t, t)

    def quad_body(p, carry):
        for s in range(NSLOT):
            t = p * NSLOT + s
            wait_gathers(t, s)
            start_store(t, s)

            @pl.when(jnp.logical_and(t >= 2, t + 2 < SEQ))
            def _(s=s, t=t):
                wait_store(t - 2, (s - 2) % NSLOT)
                start_gathers(t + 2, (s + 2) % NSLOT)

        return carry

    lax.fori_loop(0, SEQ // NSLOT, quad_body, 0)
    for t in range(SEQ - NSLOT, SEQ):
        wait_store(t, t % NSLOT)


def kernel(x, table):
    xt = x.T.astype(jnp.int32)
    return _emb_lookup(xt, table)
